# Initial kernel scaffold; baseline (speedup 1.0000x reference)
import jax
import jax.numpy as jnp
from jax.experimental import pallas as pl


def kernel(x_component, x_pin, x_subcircuit, x_net, ei_cp, ei_pc, ei_sp, ei_ps, ei_pn, ei_np, batch, nte, cte, pte, Wl, bl, Wr, C1w, C1b, C2w, C2b, C3w, C3b):
    raise NotImplementedError("write your pallas kernel here")


# speedup vs baseline: 1.0369x; 1.0369x over previous
import functools

import jax
import jax.numpy as jnp
from jax import lax
from jax.experimental import pallas as pl
from jax.experimental.pallas import tpu as pltpu

H = 256
BN = 1000


def _embed_body(p1_ref, p2_ref, p3_ref, t_ref, o_ref):
    i1 = p1_ref[0, 0, :]
    i2 = p2_ref[0, 0, :]
    i3 = p3_ref[0, 0, :]
    cols = lax.broadcasted_iota(jnp.int32, (BN, 128), 1)
    oh = ((cols == i1[:, None]).astype(jnp.float32)
          + (cols == i2[:, None]).astype(jnp.float32)
          + (cols == i3[:, None]).astype(jnp.float32))
    o_ref[...] = jnp.dot(oh, t_ref[...], preferred_element_type=jnp.float32)


def _embed(p1, p2, p3, table_pad):
    n = p1.shape[0]
    nb = n // BN
    p1 = p1.reshape(nb, 1, BN)
    p2 = p2.reshape(nb, 1, BN)
    p3 = p3.reshape(nb, 1, BN)
    return pl.pallas_call(
        _embed_body,
        grid=(nb,),
        in_specs=[
            pl.BlockSpec((1, 1, BN), lambda i: (i, 0, 0)),
            pl.BlockSpec((1, 1, BN), lambda i: (i, 0, 0)),
            pl.BlockSpec((1, 1, BN), lambda i: (i, 0, 0)),
            pl.BlockSpec((128, H), lambda i: (0, 0)),
        ],
        out_specs=pl.BlockSpec((BN, H), lambda i: (i, 0)),
        out_shape=jax.ShapeDtypeStruct((n, H), jnp.float32),
    )(p1, p2, p3, table_pad)


def _update_body(r, s_ref, inv_ref, x_ref, wl_ref, wr_ref, b_ref, o_ref):
    acc = jnp.dot(x_ref[...], wr_ref[...], preferred_element_type=jnp.float32)
    for j in range(r):
        srow = s_ref[j] * inv_ref[j, 0, 0][:, None]
        acc = acc + jnp.dot(srow, wl_ref[j], preferred_element_type=jnp.float32)
    o_ref[...] = jnp.maximum(acc + b_ref[0], 0.0)


def _update(s_stack, inv_stack, x_dst, wl_stack, wr_sum, b_sum):
    r, n, _ = s_stack.shape
    nb = n // BN
    inv4 = inv_stack.reshape(r, nb, 1, BN)
    return pl.pallas_call(
        functools.partial(_update_body, r),
        grid=(nb,),
        in_specs=[
            pl.BlockSpec((r, BN, H), lambda i: (0, i, 0)),
            pl.BlockSpec((r, 1, 1, BN), lambda i: (0, i, 0, 0)),
            pl.BlockSpec((BN, H), lambda i: (i, 0)),
            pl.BlockSpec((r, H, H), lambda i: (0, 0, 0)),
            pl.BlockSpec((H, H), lambda i: (0, 0)),
            pl.BlockSpec((1, H), lambda i: (0, 0)),
        ],
        out_specs=pl.BlockSpec((BN, H), lambda i: (i, 0)),
        out_shape=jax.ShapeDtypeStruct((n, H), jnp.float32),
    )(s_stack, inv4, x_dst, wl_stack, wr_sum, b_sum)


def _mlp_body(g_ref, w1_ref, b1_ref, w2_ref, b2_ref, w3_ref, b3_ref, o_ref):
    h = jnp.maximum(jnp.dot(g_ref[...], w1_ref[...],
                            preferred_element_type=jnp.float32) + b1_ref[0], 0.0)
    h = jnp.maximum(jnp.dot(h, w2_ref[...],
                            preferred_element_type=jnp.float32) + b2_ref[0], 0.0)
    o_ref[...] = jnp.dot(h, w3_ref[...],
                         preferred_element_type=jnp.float32) + b3_ref[0]


def _mlp(g, c1w, c1b, c2w, c2b, c3w, c3b):
    return pl.pallas_call(
        _mlp_body,
        out_shape=jax.ShapeDtypeStruct((g.shape[0], 10), jnp.float32),
    )(g, c1w, c1b.reshape(1, -1), c2w, c2b.reshape(1, -1),
      c3w, c3b.reshape(1, -1))


def _segsum(x, seg, n):
    return jax.ops.segment_sum(x, seg, num_segments=n)


def kernel(x_component, x_pin, x_subcircuit, x_net, ei_cp, ei_pc, ei_sp,
           ei_ps, ei_pn, ei_np, batch, nte, cte, pte, Wl, bl, Wr,
           C1w, C1b, C2w, C2b, C3w, C3b):
    NC = x_component.shape[0]
    NP = x_pin.shape[0]
    NS = x_subcircuit.shape[0]
    NN = x_net.shape[0]
    G = 64

    table = jnp.concatenate([nte, cte, pte], axis=0)
    table_pad = jnp.zeros((128, H), jnp.float32).at[:26].set(table)
    xs = jnp.concatenate([x_component, x_pin, x_subcircuit, x_net], axis=0)
    p1 = xs[:, 0]
    ct = jnp.clip(xs[:, 1], 0)
    ct = ct.at[:NC].set(0)
    p2 = 4 + ct
    p3 = 13 + jnp.clip(xs[:, 2], 0)
    emb = _embed(p1.astype(jnp.int32), p2.astype(jnp.int32),
                 p3.astype(jnp.int32), table_pad)
    comp = emb[:NC]
    pin = emb[NC:NC + NP]
    sub = emb[NC + NP:NC + NP + NS]
    net = emb[NC + NP + NS:]

    def inv_cnt(dst, n):
        cnt = _segsum(jnp.ones(dst.shape, jnp.float32), dst, n)
        return 1.0 / jnp.maximum(cnt, 1.0)

    inv_cp = inv_cnt(ei_cp[1], NP)
    inv_sp = inv_cnt(ei_sp[1], NP)
    inv_np = inv_cnt(ei_np[1], NP)
    inv_pc = inv_cnt(ei_pc[1], NC)
    inv_ps = inv_cnt(ei_ps[1], NS)
    inv_pn = inv_cnt(ei_pn[1], NN)

    inv_pin = jnp.stack([inv_cp, inv_sp, inv_np])

    for i in range(3):
        s_cp = _segsum(comp[ei_cp[0]], ei_cp[1], NP)
        s_sp = _segsum(sub[ei_sp[0]], ei_sp[1], NP)
        s_np = _segsum(net[ei_np[0]], ei_np[1], NP)
        s_pc = _segsum(pin[ei_pc[0]], ei_pc[1], NC)
        s_ps = _segsum(pin[ei_ps[0]], ei_ps[1], NS)
        s_pn = _segsum(pin[ei_pn[0]], ei_pn[1], NN)

        pin_new = _update(
            jnp.stack([s_cp, s_sp, s_np]), inv_pin, pin,
            jnp.stack([Wl[i, 0], Wl[i, 2], Wl[i, 5]]),
            Wr[i, 0] + Wr[i, 2] + Wr[i, 5],
            (bl[i, 0] + bl[i, 2] + bl[i, 5]).reshape(1, H))
        comp_new = _update(s_pc[None], inv_pc[None], comp, Wl[i, 1][None],
                           Wr[i, 1], bl[i, 1].reshape(1, H))
        sub_new = _update(s_ps[None], inv_ps[None], sub, Wl[i, 3][None],
                          Wr[i, 3], bl[i, 3].reshape(1, H))
        net_new = _update(s_pn[None], inv_pn[None], net, Wl[i, 4][None],
                          Wr[i, 4], bl[i, 4].reshape(1, H))
        comp, pin, sub, net = comp_new, pin_new, sub_new, net_new

    s = _segsum(comp, batch, G)
    cnt = _segsum(jnp.ones((NC,), jnp.float32), batch, G)
    mean_pool = s / jnp.maximum(cnt, 1.0)[:, None]
    mx = jax.ops.segment_max(comp, batch, num_segments=G)
    max_pool = jnp.where(jnp.isfinite(mx), mx, 0.0)
    g = jnp.concatenate([mean_pool, max_pool], axis=1)
    return _mlp(g, C1w, C1b, C2w, C2b, C3w, C3b)
